# Initial kernel scaffold; baseline (speedup 1.0000x reference)
import jax
import jax.numpy as jnp
from jax.experimental import pallas as pl


def kernel(x, edge_index, batch, W1, b1, g1, be1, W2, b2, g2, be2, W3, b3, g3, be3):
    raise NotImplementedError("write your pallas kernel here")


# speedup vs baseline: 2.2457x; 2.2457x over previous
import jax
import jax.numpy as jnp
from jax.experimental import pallas as pl

N = 10000
E = 160000
DIN = 256
DH = 256
G = 64
EPS = 1e-5


def _mm_body(x_ref, w_ref, o_ref):
    o_ref[...] = jnp.dot(x_ref[...], w_ref[...],
                         preferred_element_type=jnp.float32)


def _matmul(x, w):
    m, k = x.shape
    kk, n = w.shape
    bm = 1000
    return pl.pallas_call(
        _mm_body,
        grid=(m // bm,),
        in_specs=[pl.BlockSpec((bm, k), lambda i: (i, 0)),
                  pl.BlockSpec((kk, n), lambda i: (0, 0))],
        out_specs=pl.BlockSpec((bm, n), lambda i: (i, 0)),
        out_shape=jax.ShapeDtypeStruct((m, n), jnp.float32),
    )(x, w)


def kernel(x, edge_index, batch, W1, b1, g1, be1, W2, b2, g2, be2,
           W3, b3, g3, be3):
    src = edge_index[0]
    dst = edge_index[1]
    deg = jnp.zeros((N,), jnp.float32).at[dst].add(1.0) + 1.0
    dinv = deg ** -0.5

    def conv(h, W, b):
        hp = _matmul(h * dinv[:, None], W)
        agg = jnp.zeros((N, DH), jnp.float32).at[dst].add(hp[src]) + hp
        return dinv[:, None] * agg + b

    def bn_relu(h, g, be):
        m = jnp.mean(h, axis=0)
        v = jnp.var(h, axis=0)
        return jax.nn.relu((h - m) / jnp.sqrt(v + EPS) * g + be)

    h = bn_relu(conv(x, W1, b1), g1, be1)
    h = bn_relu(conv(h, W2, b2), g2, be2)
    h = bn_relu(conv(h, W3, b3), g3, be3)
    sums = jax.ops.segment_sum(h, batch, num_segments=G)
    cnt = jax.ops.segment_sum(jnp.ones((N,), h.dtype), batch, num_segments=G)
    mean = sums / jnp.clip(cnt, 1.0, None)[:, None]
    return jnp.concatenate([mean, sums], axis=1)


# speedup vs baseline: 6.6986x; 2.9829x over previous
import functools

import jax
import jax.numpy as jnp
from jax import lax
from jax.experimental import pallas as pl
from jax.experimental.pallas import tpu as pltpu
from jax.experimental.pallas import tpu_sc as plsc

N = 10000
E = 160000
D = 256
FH = 128
G = 64
EPS = 1e-5
NS = 16
EPT = E // NS
EPT2 = E // (2 * NS)
CH = 80
NCH = EPT // CH
CHD = 40
NCHD = EPT2 // CHD
BM = 1000

ZR_MAIN = 632
ZR_LAST = N - (NS - 1) * ZR_MAIN


def _mesh():
    return plsc.VectorSubcoreMesh(core_axis_name="c", subcore_axis_name="s",
                                  num_cores=2, num_subcores=NS)


def _tile_slab_copy(s, src_ref, dst_ref):
    @pl.when(s < NS - 1)
    def _():
        off = pl.multiple_of(s * ZR_MAIN, 8)
        pltpu.sync_copy(src_ref.at[pl.ds(off, ZR_MAIN)],
                        dst_ref.at[pl.ds(off, ZR_MAIN)])

    @pl.when(s == NS - 1)
    def _():
        pltpu.sync_copy(src_ref.at[pl.ds((NS - 1) * ZR_MAIN, ZR_LAST)],
                        dst_ref.at[pl.ds((NS - 1) * ZR_MAIN, ZR_LAST)])


def _sc_degree_call(dst, zeros128, ones128):
    @functools.partial(
        pl.kernel,
        out_type=(jax.ShapeDtypeStruct((N, FH), jnp.float32),
                  jax.ShapeDtypeStruct((N, FH), jnp.float32)),
        mesh=_mesh(),
        scratch_types=[
            pltpu.VMEM_SHARED((N, FH), jnp.float32),
            pltpu.VMEM((CHD,), jnp.int32),
            pltpu.VMEM((CHD, FH), jnp.float32),
        ],
    )
    def deg_kernel(dst_hbm, z_hbm, o_hbm, dega_hbm, degb_hbm,
                   acc, idx, ones):
        c = lax.axis_index("c")
        s = lax.axis_index("s")
        _tile_slab_copy(s, z_hbm, acc)
        pltpu.sync_copy(o_hbm.at[pl.ds(0, CHD)], ones)
        plsc.subcore_barrier()

        base = (c * NS + s) * EPT2

        def chunk(k, carry):
            off = pl.multiple_of(base + k * CHD, 8)
            pltpu.sync_copy(dst_hbm.at[pl.ds(off, CHD)], idx)
            pltpu.sync_copy(ones, acc.at[idx], add=True)
            return carry
        lax.fori_loop(0, NCHD, chunk, 0)
        plsc.subcore_barrier()

        @pl.when(c == 0)
        def _():
            _tile_slab_copy(s, acc, dega_hbm)

        @pl.when(c == 1)
        def _():
            _tile_slab_copy(s, acc, degb_hbm)

    return deg_kernel(dst, zeros128, ones128)


def _sc_agg_call(hlo, hhi, src, dst, zeros128):
    @functools.partial(
        pl.kernel,
        out_type=(jax.ShapeDtypeStruct((N, FH), jnp.float32),
                  jax.ShapeDtypeStruct((N, FH), jnp.float32)),
        mesh=_mesh(),
        scratch_types=[
            pltpu.VMEM_SHARED((N, FH), jnp.float32),
            pltpu.VMEM((CH,), jnp.int32),
            pltpu.VMEM((CH,), jnp.int32),
            pltpu.VMEM((CH, FH), jnp.float32),
            pltpu.SemaphoreType.DMA,
        ],
    )
    def agg_kernel(hlo_hbm, hhi_hbm, src_hbm, dst_hbm, z_hbm,
                   olo_hbm, ohi_hbm, acc, isrc, idst, rows, sem):
        c = lax.axis_index("c")
        s = lax.axis_index("s")
        _tile_slab_copy(s, z_hbm, acc)
        plsc.subcore_barrier()

        def run(href):
            def chunk(k, carry):
                off = pl.multiple_of(s * EPT + k * CH, 8)
                pltpu.sync_copy(src_hbm.at[pl.ds(off, CH)], isrc)
                pltpu.sync_copy(dst_hbm.at[pl.ds(off, CH)], idst)
                pltpu.async_copy(href.at[isrc], rows, sem).wait()
                pltpu.sync_copy(rows, acc.at[idst], add=True)
                return carry
            lax.fori_loop(0, NCH, chunk, 0)

        @pl.when(c == 0)
        def _():
            run(hlo_hbm)

        @pl.when(c == 1)
        def _():
            run(hhi_hbm)

        plsc.subcore_barrier()

        @pl.when(c == 0)
        def _():
            _tile_slab_copy(s, acc, olo_hbm)

        @pl.when(c == 1)
        def _():
            _tile_slab_copy(s, acc, ohi_hbm)

    return agg_kernel(hlo, hhi, src, dst, zeros128)


def _sc_pool_call(ylo, yhi, batch, zpool, ones128):
    @functools.partial(
        pl.kernel,
        out_type=(jax.ShapeDtypeStruct((G, FH), jnp.float32),
                  jax.ShapeDtypeStruct((G, FH), jnp.float32),
                  jax.ShapeDtypeStruct((G, FH), jnp.float32)),
        mesh=_mesh(),
        scratch_types=[
            pltpu.VMEM_SHARED((G, FH), jnp.float32),
            pltpu.VMEM_SHARED((G, FH), jnp.float32),
            pltpu.VMEM((CH,), jnp.int32),
            pltpu.VMEM((CH, FH), jnp.float32),
            pltpu.VMEM((CH, FH), jnp.float32),
        ],
    )
    def pool_kernel(ylo_hbm, yhi_hbm, b_hbm, zp_hbm, o_hbm,
                    plo_hbm, phi_hbm, cnt_hbm,
                    accp, accc, idx, rows, ones):
        c = lax.axis_index("c")
        s = lax.axis_index("s")

        @pl.when(s < 8)
        def _():
            off = pl.multiple_of(s * 8, 8)
            pltpu.sync_copy(zp_hbm.at[pl.ds(off, 8)], accp.at[pl.ds(off, 8)])

            @pl.when(c == 0)
            def _():
                pltpu.sync_copy(zp_hbm.at[pl.ds(off, 8)],
                                accc.at[pl.ds(off, 8)])

        pltpu.sync_copy(o_hbm.at[pl.ds(0, CH)], ones)
        plsc.subcore_barrier()

        nch = jnp.where(s == NS - 1, 5, 8)

        def run(yref, with_cnt):
            def chunk(k, carry):
                off = pl.multiple_of(s * 640 + k * CH, 8)
                pltpu.sync_copy(yref.at[pl.ds(off, CH)], rows)
                pltpu.sync_copy(b_hbm.at[pl.ds(off, CH)], idx)
                pltpu.sync_copy(rows, accp.at[idx], add=True)
                if with_cnt:
                    pltpu.sync_copy(ones, accc.at[idx], add=True)
                return carry
            lax.fori_loop(0, nch, chunk, 0)

        @pl.when(c == 0)
        def _():
            run(ylo_hbm, True)

        @pl.when(c == 1)
        def _():
            run(yhi_hbm, False)

        plsc.subcore_barrier()

        @pl.when(s < 8)
        def _():
            off = pl.multiple_of(s * 8, 8)

            @pl.when(c == 0)
            def _():
                pltpu.sync_copy(accp.at[pl.ds(off, 8)],
                                plo_hbm.at[pl.ds(off, 8)])
                pltpu.sync_copy(accc.at[pl.ds(off, 8)],
                                cnt_hbm.at[pl.ds(off, 8)])

            @pl.when(c == 1)
            def _():
                pltpu.sync_copy(accp.at[pl.ds(off, 8)],
                                phi_hbm.at[pl.ds(off, 8)])

    return pool_kernel(ylo, yhi, batch, zpool, ones128)


def _t1_body(dega_ref, degb_ref, x_ref, w_ref, dinv_ref, hlo_ref, hhi_ref):
    deg = dega_ref[...][:, :16] + degb_ref[...][:, :16]
    dinv = lax.rsqrt(deg + 1.0)
    dinv_ref[...] = dinv
    xs = x_ref[...] * dinv[:, 0:1]
    h = jnp.dot(xs, w_ref[...], preferred_element_type=jnp.float32)
    hlo_ref[...] = h[:, :FH]
    hhi_ref[...] = h[:, FH:]


def _t1_call(dega, degb, x, w):
    return pl.pallas_call(
        _t1_body,
        grid=(N // BM,),
        in_specs=[pl.BlockSpec((BM, FH), lambda i: (i, 0)),
                  pl.BlockSpec((BM, FH), lambda i: (i, 0)),
                  pl.BlockSpec((BM, D), lambda i: (i, 0)),
                  pl.BlockSpec((D, D), lambda i: (0, 0))],
        out_specs=[pl.BlockSpec((BM, 16), lambda i: (i, 0)),
                   pl.BlockSpec((BM, FH), lambda i: (i, 0)),
                   pl.BlockSpec((BM, FH), lambda i: (i, 0))],
        out_shape=[jax.ShapeDtypeStruct((N, 16), jnp.float32),
                   jax.ShapeDtypeStruct((N, FH), jnp.float32),
                   jax.ShapeDtypeStruct((N, FH), jnp.float32)],
    )(dega, degb, x, w)


def _post_body(alo_ref, ahi_ref, hlo_ref, hhi_ref, dinv_ref, b_ref,
               o_ref, st_ref, sacc):
    i = pl.program_id(0)
    dv = dinv_ref[...][:, 0:1]
    lo = alo_ref[...] + hlo_ref[...]
    hi = ahi_ref[...] + hhi_ref[...]
    o = dv * jnp.concatenate([lo, hi], axis=1) + b_ref[...]
    o_ref[...] = o

    @pl.when(i == 0)
    def _():
        sacc[...] = jnp.zeros_like(sacc)

    sacc[0:1, :] += jnp.sum(o, axis=0, keepdims=True)
    sacc[1:2, :] += jnp.sum(o * o, axis=0, keepdims=True)

    @pl.when(i == pl.num_programs(0) - 1)
    def _():
        st_ref[...] = sacc[...]


def _post_call(alo, ahi, hlo, hhi, dinv16, b2d):
    return pl.pallas_call(
        _post_body,
        grid=(N // BM,),
        in_specs=[pl.BlockSpec((BM, FH), lambda i: (i, 0)),
                  pl.BlockSpec((BM, FH), lambda i: (i, 0)),
                  pl.BlockSpec((BM, FH), lambda i: (i, 0)),
                  pl.BlockSpec((BM, FH), lambda i: (i, 0)),
                  pl.BlockSpec((BM, 16), lambda i: (i, 0)),
                  pl.BlockSpec((1, D), lambda i: (0, 0))],
        out_specs=[pl.BlockSpec((BM, D), lambda i: (i, 0)),
                   pl.BlockSpec((8, D), lambda i: (0, 0))],
        out_shape=[jax.ShapeDtypeStruct((N, D), jnp.float32),
                   jax.ShapeDtypeStruct((8, D), jnp.float32)],
        scratch_shapes=[pltpu.VMEM((8, D), jnp.float32)],
    )(alo, ahi, hlo, hhi, dinv16, b2d)


def _norm_body(o_ref, st_ref, g_ref, be_ref, dinv_ref, w_ref,
               hlo_ref, hhi_ref):
    st = st_ref[...]
    m = st[0:1, :] * (1.0 / N)
    var = st[1:2, :] * (1.0 / N) - m * m
    sc = g_ref[...] * lax.rsqrt(var + EPS)
    y = jnp.maximum((o_ref[...] - m) * sc + be_ref[...], 0.0)
    z = y * dinv_ref[...][:, 0:1]
    h = jnp.dot(z, w_ref[...], preferred_element_type=jnp.float32)
    hlo_ref[...] = h[:, :FH]
    hhi_ref[...] = h[:, FH:]


def _norm_mm_call(o, st, g2d, be2d, dinv16, w):
    return pl.pallas_call(
        _norm_body,
        grid=(N // BM,),
        in_specs=[pl.BlockSpec((BM, D), lambda i: (i, 0)),
                  pl.BlockSpec((8, D), lambda i: (0, 0)),
                  pl.BlockSpec((1, D), lambda i: (0, 0)),
                  pl.BlockSpec((1, D), lambda i: (0, 0)),
                  pl.BlockSpec((BM, 16), lambda i: (i, 0)),
                  pl.BlockSpec((D, D), lambda i: (0, 0))],
        out_specs=[pl.BlockSpec((BM, FH), lambda i: (i, 0)),
                   pl.BlockSpec((BM, FH), lambda i: (i, 0))],
        out_shape=[jax.ShapeDtypeStruct((N, FH), jnp.float32),
                   jax.ShapeDtypeStruct((N, FH), jnp.float32)],
    )(o, st, g2d, be2d, dinv16, w)


def _norm_only_body(o_ref, st_ref, g_ref, be_ref, ylo_ref, yhi_ref):
    st = st_ref[...]
    m = st[0:1, :] * (1.0 / N)
    var = st[1:2, :] * (1.0 / N) - m * m
    sc = g_ref[...] * lax.rsqrt(var + EPS)
    y = jnp.maximum((o_ref[...] - m) * sc + be_ref[...], 0.0)
    ylo_ref[...] = y[:, :FH]
    yhi_ref[...] = y[:, FH:]


def _norm_only_call(o, st, g2d, be2d):
    return pl.pallas_call(
        _norm_only_body,
        grid=(N // BM,),
        in_specs=[pl.BlockSpec((BM, D), lambda i: (i, 0)),
                  pl.BlockSpec((8, D), lambda i: (0, 0)),
                  pl.BlockSpec((1, D), lambda i: (0, 0)),
                  pl.BlockSpec((1, D), lambda i: (0, 0))],
        out_specs=[pl.BlockSpec((BM, FH), lambda i: (i, 0)),
                   pl.BlockSpec((BM, FH), lambda i: (i, 0))],
        out_shape=[jax.ShapeDtypeStruct((N, FH), jnp.float32),
                   jax.ShapeDtypeStruct((N, FH), jnp.float32)],
    )(o, st, g2d, be2d)


def _final_body(plo_ref, phi_ref, c_ref, o_ref):
    cnt = jnp.maximum(c_ref[...][:, 0:1], 1.0)
    plo = plo_ref[...]
    phi = phi_ref[...]
    o_ref[:, :FH] = plo / cnt
    o_ref[:, FH:D] = phi / cnt
    o_ref[:, D:D + FH] = plo
    o_ref[:, D + FH:] = phi


def _final_call(plo, phi, cnt):
    return pl.pallas_call(
        _final_body,
        in_specs=[pl.BlockSpec((G, FH), lambda: (0, 0)),
                  pl.BlockSpec((G, FH), lambda: (0, 0)),
                  pl.BlockSpec((G, FH), lambda: (0, 0))],
        out_specs=pl.BlockSpec((G, 2 * D), lambda: (0, 0)),
        out_shape=jax.ShapeDtypeStruct((G, 2 * D), jnp.float32),
    )(plo, phi, cnt)


def kernel(x, edge_index, batch, W1, b1, g1, be1, W2, b2, g2, be2,
           W3, b3, g3, be3):
    src = edge_index[0]
    dst = edge_index[1]
    zeros128 = jnp.zeros((N, FH), jnp.float32)
    zpool = jnp.zeros((G, FH), jnp.float32)
    ones128 = jnp.ones((CH, FH), jnp.float32)

    dega, degb = _sc_degree_call(dst, zeros128, ones128)
    dinv16, hlo, hhi = _t1_call(dega, degb, x, W1)

    for (b, g, be, wn) in ((b1, g1, be1, W2), (b2, g2, be2, W3),
                           (b3, g3, be3, None)):
        alo, ahi = _sc_agg_call(hlo, hhi, src, dst, zeros128)
        o, st = _post_call(alo, ahi, hlo, hhi, dinv16, b.reshape(1, D))
        if wn is not None:
            hlo, hhi = _norm_mm_call(o, st, g.reshape(1, D),
                                     be.reshape(1, D), dinv16, wn)
        else:
            ylo, yhi = _norm_only_call(o, st, g.reshape(1, D),
                                       be.reshape(1, D))

    plo, phi, cnt = _sc_pool_call(ylo, yhi, batch, zpool, ones128)
    return _final_call(plo, phi, cnt)


# speedup vs baseline: 11.9588x; 1.7853x over previous
import functools

import jax
import jax.numpy as jnp
from jax import lax
from jax.experimental import pallas as pl
from jax.experimental.pallas import tpu as pltpu
from jax.experimental.pallas import tpu_sc as plsc

N = 10000
E = 160000
D = 256
FH = 128
G = 64
EPS = 1e-5
NS = 16
EPT = E // NS
EPT2 = E // (2 * NS)
CH = 80
NCH = EPT // CH
CHD = 40
NCHD = EPT2 // CHD
BM = 1000

ZR_MAIN = 632
ZR_LAST = N - (NS - 1) * ZR_MAIN


def _mesh():
    return plsc.VectorSubcoreMesh(core_axis_name="c", subcore_axis_name="s",
                                  num_cores=2, num_subcores=NS)


def _tile_slab_copy(s, src_ref, dst_ref):
    @pl.when(s < NS - 1)
    def _():
        off = pl.multiple_of(s * ZR_MAIN, 8)
        pltpu.sync_copy(src_ref.at[pl.ds(off, ZR_MAIN)],
                        dst_ref.at[pl.ds(off, ZR_MAIN)])

    @pl.when(s == NS - 1)
    def _():
        pltpu.sync_copy(src_ref.at[pl.ds((NS - 1) * ZR_MAIN, ZR_LAST)],
                        dst_ref.at[pl.ds((NS - 1) * ZR_MAIN, ZR_LAST)])


def _sc_degree_call(dst, zeros128, ones128):
    @functools.partial(
        pl.kernel,
        out_type=(jax.ShapeDtypeStruct((N, FH), jnp.float32),
                  jax.ShapeDtypeStruct((N, FH), jnp.float32)),
        mesh=_mesh(),
        scratch_types=[
            pltpu.VMEM_SHARED((N, FH), jnp.float32),
            pltpu.VMEM((CHD,), jnp.int32),
            pltpu.VMEM((CHD, FH), jnp.float32),
        ],
    )
    def deg_kernel(dst_hbm, z_hbm, o_hbm, dega_hbm, degb_hbm,
                   acc, idx, ones):
        c = lax.axis_index("c")
        s = lax.axis_index("s")
        _tile_slab_copy(s, z_hbm, acc)
        pltpu.sync_copy(o_hbm.at[pl.ds(0, CHD)], ones)
        plsc.subcore_barrier()

        base = (c * NS + s) * EPT2

        def chunk(k, carry):
            off = pl.multiple_of(base + k * CHD, 8)
            pltpu.sync_copy(dst_hbm.at[pl.ds(off, CHD)], idx)
            pltpu.sync_copy(ones, acc.at[idx], add=True)
            return carry
        lax.fori_loop(0, NCHD, chunk, 0)
        plsc.subcore_barrier()

        @pl.when(c == 0)
        def _():
            _tile_slab_copy(s, acc, dega_hbm)

        @pl.when(c == 1)
        def _():
            _tile_slab_copy(s, acc, degb_hbm)

    return deg_kernel(dst, zeros128, ones128)


C2 = 128
TCH = 78


def _sc_agg_call(hlo, hhi, src, dst, zeros128):
    @functools.partial(
        pl.kernel,
        out_type=(jax.ShapeDtypeStruct((N, FH), jnp.float32),
                  jax.ShapeDtypeStruct((N, FH), jnp.float32)),
        mesh=_mesh(),
        scratch_types=[
            pltpu.VMEM_SHARED((N, FH), jnp.float32),
            pltpu.VMEM((C2,), jnp.int32),
            pltpu.VMEM((C2,), jnp.int32),
            pltpu.VMEM((C2,), jnp.int32),
            pltpu.VMEM((C2,), jnp.int32),
            pltpu.VMEM((C2, FH), jnp.float32),
            pltpu.VMEM((C2, FH), jnp.float32),
            pltpu.SemaphoreType.DMA,
            pltpu.SemaphoreType.DMA,
            pltpu.SemaphoreType.DMA,
            pltpu.SemaphoreType.DMA,
        ],
    )
    def agg_kernel(hlo_hbm, hhi_hbm, src_hbm, dst_hbm, z_hbm,
                   olo_hbm, ohi_hbm, acc,
                   isrc0, idst0, isrc1, idst1, rows0, rows1,
                   isem0, isem1, gsem0, gsem1):
        c = lax.axis_index("c")
        s = lax.axis_index("s")
        _tile_slab_copy(s, z_hbm, acc)
        plsc.subcore_barrier()

        base = s * TCH * C2
        nch = TCH + 2 * jnp.where(s == NS - 1, 1, 0)
        npair = nch // 2

        def issue_idx(isrc, idst, isem, k):
            off = pl.multiple_of(base + k * C2, 8)
            pltpu.async_copy(src_hbm.at[pl.ds(off, C2)], isrc, isem)
            pltpu.async_copy(dst_hbm.at[pl.ds(off, C2)], idst, isem)

        def wait_idx(isrc, idst, isem):
            pltpu.make_async_copy(src_hbm.at[pl.ds(0, C2)], isrc, isem).wait()
            pltpu.make_async_copy(dst_hbm.at[pl.ds(0, C2)], idst, isem).wait()

        def run(href):
            def start_gather(isrc, rows, gsem):
                pltpu.async_copy(href.at[isrc], rows, gsem)

            def wait_gather(rows, gsem):
                pltpu.make_async_copy(href.at[pl.ds(0, C2)], rows,
                                      gsem).wait()

            issue_idx(isrc0, idst0, isem0, 0)
            wait_idx(isrc0, idst0, isem0)
            start_gather(isrc0, rows0, gsem0)
            issue_idx(isrc1, idst1, isem1, 1)

            def pair(p, carry):
                wait_idx(isrc1, idst1, isem1)
                start_gather(isrc1, rows1, gsem1)
                wait_gather(rows0, gsem0)
                pltpu.sync_copy(rows0, acc.at[idst0], add=True)

                @pl.when(2 * p + 2 < nch)
                def _():
                    issue_idx(isrc0, idst0, isem0, 2 * p + 2)

                wait_gather(rows1, gsem1)
                pltpu.sync_copy(rows1, acc.at[idst1], add=True)

                @pl.when(2 * p + 3 < nch)
                def _():
                    issue_idx(isrc1, idst1, isem1, 2 * p + 3)

                @pl.when(2 * p + 2 < nch)
                def _():
                    wait_idx(isrc0, idst0, isem0)
                    start_gather(isrc0, rows0, gsem0)

                return carry
            lax.fori_loop(0, npair, pair, 0)

        @pl.when(c == 0)
        def _():
            run(hlo_hbm)

        @pl.when(c == 1)
        def _():
            run(hhi_hbm)

        plsc.subcore_barrier()

        @pl.when(c == 0)
        def _():
            _tile_slab_copy(s, acc, olo_hbm)

        @pl.when(c == 1)
        def _():
            _tile_slab_copy(s, acc, ohi_hbm)

    return agg_kernel(hlo, hhi, src, dst, zeros128)


def _sc_pool_call(ylo, yhi, batch, zpool, ones128):
    @functools.partial(
        pl.kernel,
        out_type=(jax.ShapeDtypeStruct((G, FH), jnp.float32),
                  jax.ShapeDtypeStruct((G, FH), jnp.float32),
                  jax.ShapeDtypeStruct((G, FH), jnp.float32)),
        mesh=_mesh(),
        scratch_types=[
            pltpu.VMEM_SHARED((G, FH), jnp.float32),
            pltpu.VMEM_SHARED((G, FH), jnp.float32),
            pltpu.VMEM((CH,), jnp.int32),
            pltpu.VMEM((CH, FH), jnp.float32),
            pltpu.VMEM((CH, FH), jnp.float32),
        ],
    )
    def pool_kernel(ylo_hbm, yhi_hbm, b_hbm, zp_hbm, o_hbm,
                    plo_hbm, phi_hbm, cnt_hbm,
                    accp, accc, idx, rows, ones):
        c = lax.axis_index("c")
        s = lax.axis_index("s")

        @pl.when(s < 8)
        def _():
            off = pl.multiple_of(s * 8, 8)
            pltpu.sync_copy(zp_hbm.at[pl.ds(off, 8)], accp.at[pl.ds(off, 8)])

            @pl.when(c == 0)
            def _():
                pltpu.sync_copy(zp_hbm.at[pl.ds(off, 8)],
                                accc.at[pl.ds(off, 8)])

        pltpu.sync_copy(o_hbm.at[pl.ds(0, CH)], ones)
        plsc.subcore_barrier()

        nch = jnp.where(s == NS - 1, 5, 8)

        def run(yref, with_cnt):
            def chunk(k, carry):
                off = pl.multiple_of(s * 640 + k * CH, 8)
                pltpu.sync_copy(yref.at[pl.ds(off, CH)], rows)
                pltpu.sync_copy(b_hbm.at[pl.ds(off, CH)], idx)
                pltpu.sync_copy(rows, accp.at[idx], add=True)
                if with_cnt:
                    pltpu.sync_copy(ones, accc.at[idx], add=True)
                return carry
            lax.fori_loop(0, nch, chunk, 0)

        @pl.when(c == 0)
        def _():
            run(ylo_hbm, True)

        @pl.when(c == 1)
        def _():
            run(yhi_hbm, False)

        plsc.subcore_barrier()

        @pl.when(s < 8)
        def _():
            off = pl.multiple_of(s * 8, 8)

            @pl.when(c == 0)
            def _():
                pltpu.sync_copy(accp.at[pl.ds(off, 8)],
                                plo_hbm.at[pl.ds(off, 8)])
                pltpu.sync_copy(accc.at[pl.ds(off, 8)],
                                cnt_hbm.at[pl.ds(off, 8)])

            @pl.when(c == 1)
            def _():
                pltpu.sync_copy(accp.at[pl.ds(off, 8)],
                                phi_hbm.at[pl.ds(off, 8)])

    return pool_kernel(ylo, yhi, batch, zpool, ones128)


def _t1_body(dega_ref, degb_ref, x_ref, w_ref, dinv_ref, hlo_ref, hhi_ref):
    deg = dega_ref[...][:, :16] + degb_ref[...][:, :16]
    dinv = lax.rsqrt(deg + 1.0)
    dinv_ref[...] = dinv
    xs = x_ref[...] * dinv[:, 0:1]
    h = jnp.dot(xs, w_ref[...], preferred_element_type=jnp.float32)
    hlo_ref[...] = h[:, :FH]
    hhi_ref[...] = h[:, FH:]


def _t1_call(dega, degb, x, w):
    return pl.pallas_call(
        _t1_body,
        grid=(N // BM,),
        in_specs=[pl.BlockSpec((BM, FH), lambda i: (i, 0)),
                  pl.BlockSpec((BM, FH), lambda i: (i, 0)),
                  pl.BlockSpec((BM, D), lambda i: (i, 0)),
                  pl.BlockSpec((D, D), lambda i: (0, 0))],
        out_specs=[pl.BlockSpec((BM, 16), lambda i: (i, 0)),
                   pl.BlockSpec((BM, FH), lambda i: (i, 0)),
                   pl.BlockSpec((BM, FH), lambda i: (i, 0))],
        out_shape=[jax.ShapeDtypeStruct((N, 16), jnp.float32),
                   jax.ShapeDtypeStruct((N, FH), jnp.float32),
                   jax.ShapeDtypeStruct((N, FH), jnp.float32)],
    )(dega, degb, x, w)


def _post_body(alo_ref, ahi_ref, hlo_ref, hhi_ref, dinv_ref, b_ref,
               o_ref, st_ref, sacc):
    i = pl.program_id(0)
    dv = dinv_ref[...][:, 0:1]
    lo = alo_ref[...] + hlo_ref[...]
    hi = ahi_ref[...] + hhi_ref[...]
    o = dv * jnp.concatenate([lo, hi], axis=1) + b_ref[...]
    o_ref[...] = o

    @pl.when(i == 0)
    def _():
        sacc[...] = jnp.zeros_like(sacc)

    sacc[0:1, :] += jnp.sum(o, axis=0, keepdims=True)
    sacc[1:2, :] += jnp.sum(o * o, axis=0, keepdims=True)

    @pl.when(i == pl.num_programs(0) - 1)
    def _():
        st_ref[...] = sacc[...]


def _post_call(alo, ahi, hlo, hhi, dinv16, b2d):
    return pl.pallas_call(
        _post_body,
        grid=(N // BM,),
        in_specs=[pl.BlockSpec((BM, FH), lambda i: (i, 0)),
                  pl.BlockSpec((BM, FH), lambda i: (i, 0)),
                  pl.BlockSpec((BM, FH), lambda i: (i, 0)),
                  pl.BlockSpec((BM, FH), lambda i: (i, 0)),
                  pl.BlockSpec((BM, 16), lambda i: (i, 0)),
                  pl.BlockSpec((1, D), lambda i: (0, 0))],
        out_specs=[pl.BlockSpec((BM, D), lambda i: (i, 0)),
                   pl.BlockSpec((8, D), lambda i: (0, 0))],
        out_shape=[jax.ShapeDtypeStruct((N, D), jnp.float32),
                   jax.ShapeDtypeStruct((8, D), jnp.float32)],
        scratch_shapes=[pltpu.VMEM((8, D), jnp.float32)],
    )(alo, ahi, hlo, hhi, dinv16, b2d)


def _norm_body(o_ref, st_ref, g_ref, be_ref, dinv_ref, w_ref,
               hlo_ref, hhi_ref):
    st = st_ref[...]
    m = st[0:1, :] * (1.0 / N)
    var = st[1:2, :] * (1.0 / N) - m * m
    sc = g_ref[...] * lax.rsqrt(var + EPS)
    y = jnp.maximum((o_ref[...] - m) * sc + be_ref[...], 0.0)
    z = y * dinv_ref[...][:, 0:1]
    h = jnp.dot(z, w_ref[...], preferred_element_type=jnp.float32)
    hlo_ref[...] = h[:, :FH]
    hhi_ref[...] = h[:, FH:]


def _norm_mm_call(o, st, g2d, be2d, dinv16, w):
    return pl.pallas_call(
        _norm_body,
        grid=(N // BM,),
        in_specs=[pl.BlockSpec((BM, D), lambda i: (i, 0)),
                  pl.BlockSpec((8, D), lambda i: (0, 0)),
                  pl.BlockSpec((1, D), lambda i: (0, 0)),
                  pl.BlockSpec((1, D), lambda i: (0, 0)),
                  pl.BlockSpec((BM, 16), lambda i: (i, 0)),
                  pl.BlockSpec((D, D), lambda i: (0, 0))],
        out_specs=[pl.BlockSpec((BM, FH), lambda i: (i, 0)),
                   pl.BlockSpec((BM, FH), lambda i: (i, 0))],
        out_shape=[jax.ShapeDtypeStruct((N, FH), jnp.float32),
                   jax.ShapeDtypeStruct((N, FH), jnp.float32)],
    )(o, st, g2d, be2d, dinv16, w)


def _norm_only_body(o_ref, st_ref, g_ref, be_ref, ylo_ref, yhi_ref):
    st = st_ref[...]
    m = st[0:1, :] * (1.0 / N)
    var = st[1:2, :] * (1.0 / N) - m * m
    sc = g_ref[...] * lax.rsqrt(var + EPS)
    y = jnp.maximum((o_ref[...] - m) * sc + be_ref[...], 0.0)
    ylo_ref[...] = y[:, :FH]
    yhi_ref[...] = y[:, FH:]


def _norm_only_call(o, st, g2d, be2d):
    return pl.pallas_call(
        _norm_only_body,
        grid=(N // BM,),
        in_specs=[pl.BlockSpec((BM, D), lambda i: (i, 0)),
                  pl.BlockSpec((8, D), lambda i: (0, 0)),
                  pl.BlockSpec((1, D), lambda i: (0, 0)),
                  pl.BlockSpec((1, D), lambda i: (0, 0))],
        out_specs=[pl.BlockSpec((BM, FH), lambda i: (i, 0)),
                   pl.BlockSpec((BM, FH), lambda i: (i, 0))],
        out_shape=[jax.ShapeDtypeStruct((N, FH), jnp.float32),
                   jax.ShapeDtypeStruct((N, FH), jnp.float32)],
    )(o, st, g2d, be2d)


def _final_body(plo_ref, phi_ref, c_ref, o_ref):
    cnt = jnp.maximum(c_ref[...][:, 0:1], 1.0)
    plo = plo_ref[...]
    phi = phi_ref[...]
    o_ref[:, :FH] = plo / cnt
    o_ref[:, FH:D] = phi / cnt
    o_ref[:, D:D + FH] = plo
    o_ref[:, D + FH:] = phi


def _final_call(plo, phi, cnt):
    return pl.pallas_call(
        _final_body,
        in_specs=[pl.BlockSpec((G, FH), lambda: (0, 0)),
                  pl.BlockSpec((G, FH), lambda: (0, 0)),
                  pl.BlockSpec((G, FH), lambda: (0, 0))],
        out_specs=pl.BlockSpec((G, 2 * D), lambda: (0, 0)),
        out_shape=jax.ShapeDtypeStruct((G, 2 * D), jnp.float32),
    )(plo, phi, cnt)


def kernel(x, edge_index, batch, W1, b1, g1, be1, W2, b2, g2, be2,
           W3, b3, g3, be3):
    src = edge_index[0]
    dst = edge_index[1]
    zeros128 = jnp.zeros((N, FH), jnp.float32)
    zpool = jnp.zeros((G, FH), jnp.float32)
    ones128 = jnp.ones((CH, FH), jnp.float32)

    dega, degb = _sc_degree_call(dst, zeros128, ones128)
    dinv16, hlo, hhi = _t1_call(dega, degb, x, W1)

    for (b, g, be, wn) in ((b1, g1, be1, W2), (b2, g2, be2, W3),
                           (b3, g3, be3, None)):
        alo, ahi = _sc_agg_call(hlo, hhi, src, dst, zeros128)
        o, st = _post_call(alo, ahi, hlo, hhi, dinv16, b.reshape(1, D))
        if wn is not None:
            hlo, hhi = _norm_mm_call(o, st, g.reshape(1, D),
                                     be.reshape(1, D), dinv16, wn)
        else:
            ylo, yhi = _norm_only_call(o, st, g.reshape(1, D),
                                       be.reshape(1, D))

    plo, phi, cnt = _sc_pool_call(ylo, yhi, batch, zpool, ones128)
    return _final_call(plo, phi, cnt)


# speedup vs baseline: 12.9853x; 1.0858x over previous
import functools

import jax
import jax.numpy as jnp
from jax import lax
from jax.experimental import pallas as pl
from jax.experimental.pallas import tpu as pltpu
from jax.experimental.pallas import tpu_sc as plsc

N = 10000
E = 160000
D = 256
FH = 128
G = 64
EPS = 1e-5
NS = 16
EPT = E // NS
EPT2 = E // (2 * NS)
CH = 80
NCH = EPT // CH
CHD = 40
NCHD = EPT2 // CHD
BM = 1000

ZR_MAIN = 632
ZR_LAST = N - (NS - 1) * ZR_MAIN


def _mesh():
    return plsc.VectorSubcoreMesh(core_axis_name="c", subcore_axis_name="s",
                                  num_cores=2, num_subcores=NS)


def _tile_slab_copy(s, src_ref, dst_ref):
    @pl.when(s < NS - 1)
    def _():
        off = pl.multiple_of(s * ZR_MAIN, 8)
        pltpu.sync_copy(src_ref.at[pl.ds(off, ZR_MAIN)],
                        dst_ref.at[pl.ds(off, ZR_MAIN)])

    @pl.when(s == NS - 1)
    def _():
        pltpu.sync_copy(src_ref.at[pl.ds((NS - 1) * ZR_MAIN, ZR_LAST)],
                        dst_ref.at[pl.ds((NS - 1) * ZR_MAIN, ZR_LAST)])


DW = 16
TCHD = 39


def _sc_degree_call(dst, zeros128, ones_deg):
    @functools.partial(
        pl.kernel,
        out_type=(jax.ShapeDtypeStruct((N, FH), jnp.float32),
                  jax.ShapeDtypeStruct((N, FH), jnp.float32)),
        mesh=_mesh(),
        scratch_types=[
            pltpu.VMEM_SHARED((N, FH), jnp.float32),
            pltpu.VMEM((C2,), jnp.int32),
            pltpu.VMEM((C2,), jnp.int32),
            pltpu.VMEM((C2, FH), jnp.float32),
            pltpu.SemaphoreType.DMA,
            pltpu.SemaphoreType.DMA,
        ],
    )
    def deg_kernel(dst_hbm, z_hbm, o_hbm, dega_hbm, degb_hbm,
                   acc, idx0, idx1, ones, sem0, sem1):
        c = lax.axis_index("c")
        s = lax.axis_index("s")
        _tile_slab_copy(s, z_hbm, acc)
        pltpu.sync_copy(o_hbm, ones)

        base = c * (E // 2) + s * TCHD * C2
        nch = TCHD + jnp.where(s == NS - 1, 1, 0)
        npair = (nch + 1) // 2

        def issue(idx, sem, k):
            off = pl.multiple_of(base + k * C2, 8)
            pltpu.async_copy(dst_hbm.at[pl.ds(off, C2)], idx, sem)

        def wait(idx, sem):
            pltpu.make_async_copy(dst_hbm.at[pl.ds(0, C2)], idx, sem).wait()

        plsc.subcore_barrier()

        issue(idx0, sem0, 0)
        issue(idx1, sem1, 1)

        def pair(p, carry):
            wait(idx0, sem0)
            pltpu.sync_copy(ones, acc.at[idx0], add=True)

            @pl.when(2 * p + 2 < nch)
            def _():
                issue(idx0, sem0, 2 * p + 2)

            @pl.when(2 * p + 1 < nch)
            def _():
                wait(idx1, sem1)
                pltpu.sync_copy(ones, acc.at[idx1], add=True)

            @pl.when(2 * p + 3 < nch)
            def _():
                issue(idx1, sem1, 2 * p + 3)

            return carry
        lax.fori_loop(0, npair, pair, 0)
        plsc.subcore_barrier()

        @pl.when(c == 0)
        def _():
            _tile_slab_copy(s, acc, dega_hbm)

        @pl.when(c == 1)
        def _():
            _tile_slab_copy(s, acc, degb_hbm)

    return deg_kernel(dst, zeros128, ones_deg)


C2 = 128
TCH = 78


def _sc_agg_call(hlo, hhi, src, dst, zeros128):
    @functools.partial(
        pl.kernel,
        out_type=(jax.ShapeDtypeStruct((N, FH), jnp.float32),
                  jax.ShapeDtypeStruct((N, FH), jnp.float32)),
        mesh=_mesh(),
        scratch_types=[
            pltpu.VMEM_SHARED((N, FH), jnp.float32),
            pltpu.VMEM((C2,), jnp.int32),
            pltpu.VMEM((C2,), jnp.int32),
            pltpu.VMEM((C2,), jnp.int32),
            pltpu.VMEM((C2,), jnp.int32),
            pltpu.VMEM((C2, FH), jnp.float32),
            pltpu.VMEM((C2, FH), jnp.float32),
            pltpu.SemaphoreType.DMA,
            pltpu.SemaphoreType.DMA,
            pltpu.SemaphoreType.DMA,
            pltpu.SemaphoreType.DMA,
        ],
    )
    def agg_kernel(hlo_hbm, hhi_hbm, src_hbm, dst_hbm, z_hbm,
                   olo_hbm, ohi_hbm, acc,
                   isrc0, idst0, isrc1, idst1, rows0, rows1,
                   isem0, isem1, gsem0, gsem1):
        c = lax.axis_index("c")
        s = lax.axis_index("s")
        _tile_slab_copy(s, z_hbm, acc)
        plsc.subcore_barrier()

        base = s * TCH * C2
        nch = TCH + 2 * jnp.where(s == NS - 1, 1, 0)
        npair = nch // 2

        def issue_idx(isrc, idst, isem, k):
            off = pl.multiple_of(base + k * C2, 8)
            pltpu.async_copy(src_hbm.at[pl.ds(off, C2)], isrc, isem)
            pltpu.async_copy(dst_hbm.at[pl.ds(off, C2)], idst, isem)

        def wait_idx(isrc, idst, isem):
            pltpu.make_async_copy(src_hbm.at[pl.ds(0, C2)], isrc, isem).wait()
            pltpu.make_async_copy(dst_hbm.at[pl.ds(0, C2)], idst, isem).wait()

        def run(href):
            def start_gather(isrc, rows, gsem):
                pltpu.async_copy(href.at[isrc], rows, gsem)

            def wait_gather(rows, gsem):
                pltpu.make_async_copy(href.at[pl.ds(0, C2)], rows,
                                      gsem).wait()

            issue_idx(isrc0, idst0, isem0, 0)
            wait_idx(isrc0, idst0, isem0)
            start_gather(isrc0, rows0, gsem0)
            issue_idx(isrc1, idst1, isem1, 1)

            def pair(p, carry):
                wait_idx(isrc1, idst1, isem1)
                start_gather(isrc1, rows1, gsem1)
                wait_gather(rows0, gsem0)
                pltpu.sync_copy(rows0, acc.at[idst0], add=True)

                @pl.when(2 * p + 2 < nch)
                def _():
                    issue_idx(isrc0, idst0, isem0, 2 * p + 2)

                wait_gather(rows1, gsem1)
                pltpu.sync_copy(rows1, acc.at[idst1], add=True)

                @pl.when(2 * p + 3 < nch)
                def _():
                    issue_idx(isrc1, idst1, isem1, 2 * p + 3)

                @pl.when(2 * p + 2 < nch)
                def _():
                    wait_idx(isrc0, idst0, isem0)
                    start_gather(isrc0, rows0, gsem0)

                return carry
            lax.fori_loop(0, npair, pair, 0)

        @pl.when(c == 0)
        def _():
            run(hlo_hbm)

        @pl.when(c == 1)
        def _():
            run(hhi_hbm)

        plsc.subcore_barrier()

        @pl.when(c == 0)
        def _():
            _tile_slab_copy(s, acc, olo_hbm)

        @pl.when(c == 1)
        def _():
            _tile_slab_copy(s, acc, ohi_hbm)

    return agg_kernel(hlo, hhi, src, dst, zeros128)


def _sc_pool_call(ylo, yhi, batch, zpool, ones128):
    @functools.partial(
        pl.kernel,
        out_type=(jax.ShapeDtypeStruct((G, FH), jnp.float32),
                  jax.ShapeDtypeStruct((G, FH), jnp.float32),
                  jax.ShapeDtypeStruct((G, FH), jnp.float32)),
        mesh=_mesh(),
        scratch_types=[
            pltpu.VMEM_SHARED((G, FH), jnp.float32),
            pltpu.VMEM_SHARED((G, FH), jnp.float32),
            pltpu.VMEM((CH,), jnp.int32),
            pltpu.VMEM((CH, FH), jnp.float32),
            pltpu.VMEM((CH, FH), jnp.float32),
        ],
    )
    def pool_kernel(ylo_hbm, yhi_hbm, b_hbm, zp_hbm, o_hbm,
                    plo_hbm, phi_hbm, cnt_hbm,
                    accp, accc, idx, rows, ones):
        c = lax.axis_index("c")
        s = lax.axis_index("s")

        @pl.when(s < 8)
        def _():
            off = pl.multiple_of(s * 8, 8)
            pltpu.sync_copy(zp_hbm.at[pl.ds(off, 8)], accp.at[pl.ds(off, 8)])

            @pl.when(c == 0)
            def _():
                pltpu.sync_copy(zp_hbm.at[pl.ds(off, 8)],
                                accc.at[pl.ds(off, 8)])

        pltpu.sync_copy(o_hbm.at[pl.ds(0, CH)], ones)
        plsc.subcore_barrier()

        nch = jnp.where(s == NS - 1, 5, 8)

        def run(yref, with_cnt):
            def chunk(k, carry):
                off = pl.multiple_of(s * 640 + k * CH, 8)
                pltpu.sync_copy(yref.at[pl.ds(off, CH)], rows)
                pltpu.sync_copy(b_hbm.at[pl.ds(off, CH)], idx)
                pltpu.sync_copy(rows, accp.at[idx], add=True)
                if with_cnt:
                    pltpu.sync_copy(ones, accc.at[idx], add=True)
                return carry
            lax.fori_loop(0, nch, chunk, 0)

        @pl.when(c == 0)
        def _():
            run(ylo_hbm, True)

        @pl.when(c == 1)
        def _():
            run(yhi_hbm, False)

        plsc.subcore_barrier()

        @pl.when(s < 8)
        def _():
            off = pl.multiple_of(s * 8, 8)

            @pl.when(c == 0)
            def _():
                pltpu.sync_copy(accp.at[pl.ds(off, 8)],
                                plo_hbm.at[pl.ds(off, 8)])
                pltpu.sync_copy(accc.at[pl.ds(off, 8)],
                                cnt_hbm.at[pl.ds(off, 8)])

            @pl.when(c == 1)
            def _():
                pltpu.sync_copy(accp.at[pl.ds(off, 8)],
                                phi_hbm.at[pl.ds(off, 8)])

    return pool_kernel(ylo, yhi, batch, zpool, ones128)


def _t1_body(dega_ref, degb_ref, x_ref, w_ref, dinv_ref, hlo_ref, hhi_ref):
    deg = dega_ref[...] + degb_ref[...]
    dinv = lax.rsqrt(deg + 1.0)
    dinv_ref[...] = dinv[:, :DW]
    xs = x_ref[...] * dinv[:, 0:1]
    h = jnp.dot(xs, w_ref[...], preferred_element_type=jnp.float32)
    hlo_ref[...] = h[:, :FH]
    hhi_ref[...] = h[:, FH:]


def _t1_call(dega, degb, x, w):
    return pl.pallas_call(
        _t1_body,
        grid=(N // BM,),
        in_specs=[pl.BlockSpec((BM, FH), lambda i: (i, 0)),
                  pl.BlockSpec((BM, FH), lambda i: (i, 0)),
                  pl.BlockSpec((BM, D), lambda i: (i, 0)),
                  pl.BlockSpec((D, D), lambda i: (0, 0))],
        out_specs=[pl.BlockSpec((BM, 16), lambda i: (i, 0)),
                   pl.BlockSpec((BM, FH), lambda i: (i, 0)),
                   pl.BlockSpec((BM, FH), lambda i: (i, 0))],
        out_shape=[jax.ShapeDtypeStruct((N, 16), jnp.float32),
                   jax.ShapeDtypeStruct((N, FH), jnp.float32),
                   jax.ShapeDtypeStruct((N, FH), jnp.float32)],
    )(dega, degb, x, w)


def _post_body(alo_ref, ahi_ref, hlo_ref, hhi_ref, dinv_ref, b_ref,
               o_ref, st_ref, sacc):
    i = pl.program_id(0)
    dv = dinv_ref[...][:, 0:1]
    lo = alo_ref[...] + hlo_ref[...]
    hi = ahi_ref[...] + hhi_ref[...]
    o = dv * jnp.concatenate([lo, hi], axis=1) + b_ref[...]
    o_ref[...] = o

    @pl.when(i == 0)
    def _():
        sacc[...] = jnp.zeros_like(sacc)

    sacc[0:1, :] += jnp.sum(o, axis=0, keepdims=True)
    sacc[1:2, :] += jnp.sum(o * o, axis=0, keepdims=True)

    @pl.when(i == pl.num_programs(0) - 1)
    def _():
        st_ref[...] = sacc[...]


def _post_call(alo, ahi, hlo, hhi, dinv16, b2d):
    return pl.pallas_call(
        _post_body,
        grid=(N // BM,),
        in_specs=[pl.BlockSpec((BM, FH), lambda i: (i, 0)),
                  pl.BlockSpec((BM, FH), lambda i: (i, 0)),
                  pl.BlockSpec((BM, FH), lambda i: (i, 0)),
                  pl.BlockSpec((BM, FH), lambda i: (i, 0)),
                  pl.BlockSpec((BM, 16), lambda i: (i, 0)),
                  pl.BlockSpec((1, D), lambda i: (0, 0))],
        out_specs=[pl.BlockSpec((BM, D), lambda i: (i, 0)),
                   pl.BlockSpec((8, D), lambda i: (0, 0))],
        out_shape=[jax.ShapeDtypeStruct((N, D), jnp.float32),
                   jax.ShapeDtypeStruct((8, D), jnp.float32)],
        scratch_shapes=[pltpu.VMEM((8, D), jnp.float32)],
    )(alo, ahi, hlo, hhi, dinv16, b2d)


def _norm_body(o_ref, st_ref, g_ref, be_ref, dinv_ref, w_ref,
               hlo_ref, hhi_ref):
    st = st_ref[...]
    m = st[0:1, :] * (1.0 / N)
    var = st[1:2, :] * (1.0 / N) - m * m
    sc = g_ref[...] * lax.rsqrt(var + EPS)
    y = jnp.maximum((o_ref[...] - m) * sc + be_ref[...], 0.0)
    z = y * dinv_ref[...][:, 0:1]
    h = jnp.dot(z, w_ref[...], preferred_element_type=jnp.float32)
    hlo_ref[...] = h[:, :FH]
    hhi_ref[...] = h[:, FH:]


def _norm_mm_call(o, st, g2d, be2d, dinv16, w):
    return pl.pallas_call(
        _norm_body,
        grid=(N // BM,),
        in_specs=[pl.BlockSpec((BM, D), lambda i: (i, 0)),
                  pl.BlockSpec((8, D), lambda i: (0, 0)),
                  pl.BlockSpec((1, D), lambda i: (0, 0)),
                  pl.BlockSpec((1, D), lambda i: (0, 0)),
                  pl.BlockSpec((BM, 16), lambda i: (i, 0)),
                  pl.BlockSpec((D, D), lambda i: (0, 0))],
        out_specs=[pl.BlockSpec((BM, FH), lambda i: (i, 0)),
                   pl.BlockSpec((BM, FH), lambda i: (i, 0))],
        out_shape=[jax.ShapeDtypeStruct((N, FH), jnp.float32),
                   jax.ShapeDtypeStruct((N, FH), jnp.float32)],
    )(o, st, g2d, be2d, dinv16, w)


def _norm_only_body(o_ref, st_ref, g_ref, be_ref, ylo_ref, yhi_ref):
    st = st_ref[...]
    m = st[0:1, :] * (1.0 / N)
    var = st[1:2, :] * (1.0 / N) - m * m
    sc = g_ref[...] * lax.rsqrt(var + EPS)
    y = jnp.maximum((o_ref[...] - m) * sc + be_ref[...], 0.0)
    ylo_ref[...] = y[:, :FH]
    yhi_ref[...] = y[:, FH:]


def _norm_only_call(o, st, g2d, be2d):
    return pl.pallas_call(
        _norm_only_body,
        grid=(N // BM,),
        in_specs=[pl.BlockSpec((BM, D), lambda i: (i, 0)),
                  pl.BlockSpec((8, D), lambda i: (0, 0)),
                  pl.BlockSpec((1, D), lambda i: (0, 0)),
                  pl.BlockSpec((1, D), lambda i: (0, 0))],
        out_specs=[pl.BlockSpec((BM, FH), lambda i: (i, 0)),
                   pl.BlockSpec((BM, FH), lambda i: (i, 0))],
        out_shape=[jax.ShapeDtypeStruct((N, FH), jnp.float32),
                   jax.ShapeDtypeStruct((N, FH), jnp.float32)],
    )(o, st, g2d, be2d)


def _final_body(plo_ref, phi_ref, c_ref, o_ref):
    cnt = jnp.maximum(c_ref[...][:, 0:1], 1.0)
    plo = plo_ref[...]
    phi = phi_ref[...]
    o_ref[:, :FH] = plo / cnt
    o_ref[:, FH:D] = phi / cnt
    o_ref[:, D:D + FH] = plo
    o_ref[:, D + FH:] = phi


def _final_call(plo, phi, cnt):
    return pl.pallas_call(
        _final_body,
        in_specs=[pl.BlockSpec((G, FH), lambda: (0, 0)),
                  pl.BlockSpec((G, FH), lambda: (0, 0)),
                  pl.BlockSpec((G, FH), lambda: (0, 0))],
        out_specs=pl.BlockSpec((G, 2 * D), lambda: (0, 0)),
        out_shape=jax.ShapeDtypeStruct((G, 2 * D), jnp.float32),
    )(plo, phi, cnt)


def kernel(x, edge_index, batch, W1, b1, g1, be1, W2, b2, g2, be2,
           W3, b3, g3, be3):
    src = edge_index[0]
    dst = edge_index[1]
    zeros128 = jnp.zeros((N, FH), jnp.float32)
    zpool = jnp.zeros((G, FH), jnp.float32)
    ones128 = jnp.ones((CH, FH), jnp.float32)
    ones_deg = jnp.ones((C2, FH), jnp.float32)

    dega, degb = _sc_degree_call(dst, zeros128, ones_deg)
    dinv16, hlo, hhi = _t1_call(dega, degb, x, W1)

    for (b, g, be, wn) in ((b1, g1, be1, W2), (b2, g2, be2, W3),
                           (b3, g3, be3, None)):
        alo, ahi = _sc_agg_call(hlo, hhi, src, dst, zeros128)
        o, st = _post_call(alo, ahi, hlo, hhi, dinv16, b.reshape(1, D))
        if wn is not None:
            hlo, hhi = _norm_mm_call(o, st, g.reshape(1, D),
                                     be.reshape(1, D), dinv16, wn)
        else:
            ylo, yhi = _norm_only_call(o, st, g.reshape(1, D),
                                       be.reshape(1, D))

    plo, phi, cnt = _sc_pool_call(ylo, yhi, batch, zpool, ones128)
    return _final_call(plo, phi, cnt)


# speedup vs baseline: 14.4647x; 1.1139x over previous
import functools

import jax
import jax.numpy as jnp
from jax import lax
from jax.experimental import pallas as pl
from jax.experimental.pallas import tpu as pltpu
from jax.experimental.pallas import tpu_sc as plsc

N = 10000
E = 160000
D = 256
FH = 128
G = 64
EPS = 1e-5
NS = 16
EPT = E // NS
EPT2 = E // (2 * NS)
CH = 80
NCH = EPT // CH
CHD = 40
NCHD = EPT2 // CHD
BM = 1000

ZR_MAIN = 632
ZR_LAST = N - (NS - 1) * ZR_MAIN


def _mesh():
    return plsc.VectorSubcoreMesh(core_axis_name="c", subcore_axis_name="s",
                                  num_cores=2, num_subcores=NS)


def _tile_slab_copy(s, src_ref, dst_ref):
    @pl.when(s < NS - 1)
    def _():
        off = pl.multiple_of(s * ZR_MAIN, 8)
        pltpu.sync_copy(src_ref.at[pl.ds(off, ZR_MAIN)],
                        dst_ref.at[pl.ds(off, ZR_MAIN)])

    @pl.when(s == NS - 1)
    def _():
        pltpu.sync_copy(src_ref.at[pl.ds((NS - 1) * ZR_MAIN, ZR_LAST)],
                        dst_ref.at[pl.ds((NS - 1) * ZR_MAIN, ZR_LAST)])


DW = 16
TCHD = 39


def _sc_degree_call(dst, zeros128, ones_deg):
    @functools.partial(
        pl.kernel,
        out_type=(jax.ShapeDtypeStruct((N, FH), jnp.float32),
                  jax.ShapeDtypeStruct((N, FH), jnp.float32)),
        mesh=_mesh(),
        scratch_types=[
            pltpu.VMEM_SHARED((N, FH), jnp.float32),
            pltpu.VMEM((C2,), jnp.int32),
            pltpu.VMEM((C2,), jnp.int32),
            pltpu.VMEM((C2, FH), jnp.float32),
            pltpu.SemaphoreType.DMA,
            pltpu.SemaphoreType.DMA,
        ],
    )
    def deg_kernel(dst_hbm, z_hbm, o_hbm, dega_hbm, degb_hbm,
                   acc, idx0, idx1, ones, sem0, sem1):
        c = lax.axis_index("c")
        s = lax.axis_index("s")
        _tile_slab_copy(s, z_hbm, acc)
        pltpu.sync_copy(o_hbm, ones)

        base = c * (E // 2) + s * TCHD * C2
        nch = TCHD + jnp.where(s == NS - 1, 1, 0)
        npair = (nch + 1) // 2

        def issue(idx, sem, k):
            off = pl.multiple_of(base + k * C2, 8)
            pltpu.async_copy(dst_hbm.at[pl.ds(off, C2)], idx, sem)

        def wait(idx, sem):
            pltpu.make_async_copy(dst_hbm.at[pl.ds(0, C2)], idx, sem).wait()

        plsc.subcore_barrier()

        issue(idx0, sem0, 0)
        issue(idx1, sem1, 1)

        def pair(p, carry):
            wait(idx0, sem0)
            pltpu.sync_copy(ones, acc.at[idx0], add=True)

            @pl.when(2 * p + 2 < nch)
            def _():
                issue(idx0, sem0, 2 * p + 2)

            @pl.when(2 * p + 1 < nch)
            def _():
                wait(idx1, sem1)
                pltpu.sync_copy(ones, acc.at[idx1], add=True)

            @pl.when(2 * p + 3 < nch)
            def _():
                issue(idx1, sem1, 2 * p + 3)

            return carry
        lax.fori_loop(0, npair, pair, 0)
        plsc.subcore_barrier()

        @pl.when(c == 0)
        def _():
            _tile_slab_copy(s, acc, dega_hbm)

        @pl.when(c == 1)
        def _():
            _tile_slab_copy(s, acc, degb_hbm)

    return deg_kernel(dst, zeros128, ones_deg)


C2 = 128
TCH = 78


def _sc_agg_call(hlo, hhi, src, dst, zeros128):
    @functools.partial(
        pl.kernel,
        out_type=(jax.ShapeDtypeStruct((N, FH), jnp.float32),
                  jax.ShapeDtypeStruct((N, FH), jnp.float32)),
        mesh=_mesh(),
        scratch_types=[
            pltpu.VMEM_SHARED((N, FH), jnp.float32),
            pltpu.VMEM((C2,), jnp.int32),
            pltpu.VMEM((C2,), jnp.int32),
            pltpu.VMEM((C2,), jnp.int32),
            pltpu.VMEM((C2,), jnp.int32),
            pltpu.VMEM((C2,), jnp.int32),
            pltpu.VMEM((C2,), jnp.int32),
            pltpu.VMEM((C2, FH), jnp.float32),
            pltpu.VMEM((C2, FH), jnp.float32),
            pltpu.VMEM((C2, FH), jnp.float32),
            pltpu.SemaphoreType.DMA,
            pltpu.SemaphoreType.DMA,
            pltpu.SemaphoreType.DMA,
            pltpu.SemaphoreType.DMA,
            pltpu.SemaphoreType.DMA,
            pltpu.SemaphoreType.DMA,
        ],
    )
    def agg_kernel(hlo_hbm, hhi_hbm, src_hbm, dst_hbm, z_hbm,
                   olo_hbm, ohi_hbm, acc,
                   isrc0, idst0, isrc1, idst1, isrc2, idst2,
                   rows0, rows1, rows2,
                   isem0, isem1, isem2, gsem0, gsem1, gsem2):
        c = lax.axis_index("c")
        s = lax.axis_index("s")
        _tile_slab_copy(s, z_hbm, acc)
        plsc.subcore_barrier()

        base = s * TCH * C2
        nch = TCH + 2 * jnp.where(s == NS - 1, 1, 0)
        ntrip = (nch + 2) // 3

        bufs = ((isrc0, idst0, isem0, rows0, gsem0),
                (isrc1, idst1, isem1, rows1, gsem1),
                (isrc2, idst2, isem2, rows2, gsem2))

        def issue_idx(b, k):
            isrc, idst, isem, _, _ = b
            off = pl.multiple_of(base + k * C2, 8)
            pltpu.async_copy(src_hbm.at[pl.ds(off, C2)], isrc, isem)
            pltpu.async_copy(dst_hbm.at[pl.ds(off, C2)], idst, isem)

        def wait_idx(b):
            isrc, idst, isem, _, _ = b
            pltpu.make_async_copy(src_hbm.at[pl.ds(0, C2)], isrc, isem).wait()
            pltpu.make_async_copy(dst_hbm.at[pl.ds(0, C2)], idst, isem).wait()

        def run(href):
            def start_gather(b):
                isrc, _, _, rows, gsem = b
                pltpu.async_copy(href.at[isrc], rows, gsem)

            def wait_gather(b):
                _, _, _, rows, gsem = b
                pltpu.make_async_copy(href.at[pl.ds(0, C2)], rows,
                                      gsem).wait()

            def scatter(b):
                _, idst, _, rows, _ = b
                pltpu.sync_copy(rows, acc.at[idst], add=True)

            issue_idx(bufs[0], 0)
            issue_idx(bufs[1], 1)
            issue_idx(bufs[2], 2)
            wait_idx(bufs[0])
            start_gather(bufs[0])
            wait_idx(bufs[1])
            start_gather(bufs[1])

            def trip(t, carry):
                k = 3 * t
                wait_gather(bufs[0])

                @pl.when(k + 2 < nch)
                def _():
                    wait_idx(bufs[2])
                    start_gather(bufs[2])

                scatter(bufs[0])

                @pl.when(k + 3 < nch)
                def _():
                    issue_idx(bufs[0], k + 3)

                @pl.when(k + 1 < nch)
                def _():
                    wait_gather(bufs[1])

                    @pl.when(k + 3 < nch)
                    def _():
                        wait_idx(bufs[0])
                        start_gather(bufs[0])

                    scatter(bufs[1])

                    @pl.when(k + 4 < nch)
                    def _():
                        issue_idx(bufs[1], k + 4)

                @pl.when(k + 2 < nch)
                def _():
                    wait_gather(bufs[2])

                    @pl.when(k + 4 < nch)
                    def _():
                        wait_idx(bufs[1])
                        start_gather(bufs[1])

                    scatter(bufs[2])

                    @pl.when(k + 5 < nch)
                    def _():
                        issue_idx(bufs[2], k + 5)

                return carry
            lax.fori_loop(0, ntrip, trip, 0)

        @pl.when(c == 0)
        def _():
            run(hlo_hbm)

        @pl.when(c == 1)
        def _():
            run(hhi_hbm)

        plsc.subcore_barrier()

        @pl.when(c == 0)
        def _():
            _tile_slab_copy(s, acc, olo_hbm)

        @pl.when(c == 1)
        def _():
            _tile_slab_copy(s, acc, ohi_hbm)

    return agg_kernel(hlo, hhi, src, dst, zeros128)


def _sc_pool_call(ylo, yhi, batch, zpool, ones128):
    @functools.partial(
        pl.kernel,
        out_type=(jax.ShapeDtypeStruct((G, FH), jnp.float32),
                  jax.ShapeDtypeStruct((G, FH), jnp.float32),
                  jax.ShapeDtypeStruct((G, FH), jnp.float32)),
        mesh=_mesh(),
        scratch_types=[
            pltpu.VMEM_SHARED((G, FH), jnp.float32),
            pltpu.VMEM_SHARED((G, FH), jnp.float32),
            pltpu.VMEM((CH,), jnp.int32),
            pltpu.VMEM((CH, FH), jnp.float32),
            pltpu.VMEM((CH, FH), jnp.float32),
        ],
    )
    def pool_kernel(ylo_hbm, yhi_hbm, b_hbm, zp_hbm, o_hbm,
                    plo_hbm, phi_hbm, cnt_hbm,
                    accp, accc, idx, rows, ones):
        c = lax.axis_index("c")
        s = lax.axis_index("s")

        @pl.when(s < 8)
        def _():
            off = pl.multiple_of(s * 8, 8)
            pltpu.sync_copy(zp_hbm.at[pl.ds(off, 8)], accp.at[pl.ds(off, 8)])

            @pl.when(c == 0)
            def _():
                pltpu.sync_copy(zp_hbm.at[pl.ds(off, 8)],
                                accc.at[pl.ds(off, 8)])

        pltpu.sync_copy(o_hbm.at[pl.ds(0, CH)], ones)
        plsc.subcore_barrier()

        nch = jnp.where(s == NS - 1, 5, 8)

        def run(yref, with_cnt):
            def chunk(k, carry):
                off = pl.multiple_of(s * 640 + k * CH, 8)
                pltpu.sync_copy(yref.at[pl.ds(off, CH)], rows)
                pltpu.sync_copy(b_hbm.at[pl.ds(off, CH)], idx)
                pltpu.sync_copy(rows, accp.at[idx], add=True)
                if with_cnt:
                    pltpu.sync_copy(ones, accc.at[idx], add=True)
                return carry
            lax.fori_loop(0, nch, chunk, 0)

        @pl.when(c == 0)
        def _():
            run(ylo_hbm, True)

        @pl.when(c == 1)
        def _():
            run(yhi_hbm, False)

        plsc.subcore_barrier()

        @pl.when(s < 8)
        def _():
            off = pl.multiple_of(s * 8, 8)

            @pl.when(c == 0)
            def _():
                pltpu.sync_copy(accp.at[pl.ds(off, 8)],
                                plo_hbm.at[pl.ds(off, 8)])
                pltpu.sync_copy(accc.at[pl.ds(off, 8)],
                                cnt_hbm.at[pl.ds(off, 8)])

            @pl.when(c == 1)
            def _():
                pltpu.sync_copy(accp.at[pl.ds(off, 8)],
                                phi_hbm.at[pl.ds(off, 8)])

    return pool_kernel(ylo, yhi, batch, zpool, ones128)


def _t1_body(dega_ref, degb_ref, x_ref, w_ref, dinv_ref, hlo_ref, hhi_ref):
    deg = dega_ref[...] + degb_ref[...]
    dinv = lax.rsqrt(deg + 1.0)
    dinv_ref[...] = dinv[:, :DW]
    xs = x_ref[...] * dinv[:, 0:1]
    h = jnp.dot(xs, w_ref[...], preferred_element_type=jnp.float32)
    hlo_ref[...] = h[:, :FH]
    hhi_ref[...] = h[:, FH:]


def _t1_call(dega, degb, x, w):
    return pl.pallas_call(
        _t1_body,
        grid=(N // BM,),
        in_specs=[pl.BlockSpec((BM, FH), lambda i: (i, 0)),
                  pl.BlockSpec((BM, FH), lambda i: (i, 0)),
                  pl.BlockSpec((BM, D), lambda i: (i, 0)),
                  pl.BlockSpec((D, D), lambda i: (0, 0))],
        out_specs=[pl.BlockSpec((BM, 16), lambda i: (i, 0)),
                   pl.BlockSpec((BM, FH), lambda i: (i, 0)),
                   pl.BlockSpec((BM, FH), lambda i: (i, 0))],
        out_shape=[jax.ShapeDtypeStruct((N, 16), jnp.float32),
                   jax.ShapeDtypeStruct((N, FH), jnp.float32),
                   jax.ShapeDtypeStruct((N, FH), jnp.float32)],
    )(dega, degb, x, w)


def _post_body(alo_ref, ahi_ref, hlo_ref, hhi_ref, dinv_ref, b_ref,
               o_ref, st_ref, sacc):
    i = pl.program_id(0)
    dv = dinv_ref[...][:, 0:1]
    lo = alo_ref[...] + hlo_ref[...]
    hi = ahi_ref[...] + hhi_ref[...]
    o = dv * jnp.concatenate([lo, hi], axis=1) + b_ref[...]
    o_ref[...] = o

    @pl.when(i == 0)
    def _():
        sacc[...] = jnp.zeros_like(sacc)

    sacc[0:1, :] += jnp.sum(o, axis=0, keepdims=True)
    sacc[1:2, :] += jnp.sum(o * o, axis=0, keepdims=True)

    @pl.when(i == pl.num_programs(0) - 1)
    def _():
        st_ref[...] = sacc[...]


def _post_call(alo, ahi, hlo, hhi, dinv16, b2d):
    return pl.pallas_call(
        _post_body,
        grid=(N // BM,),
        in_specs=[pl.BlockSpec((BM, FH), lambda i: (i, 0)),
                  pl.BlockSpec((BM, FH), lambda i: (i, 0)),
                  pl.BlockSpec((BM, FH), lambda i: (i, 0)),
                  pl.BlockSpec((BM, FH), lambda i: (i, 0)),
                  pl.BlockSpec((BM, 16), lambda i: (i, 0)),
                  pl.BlockSpec((1, D), lambda i: (0, 0))],
        out_specs=[pl.BlockSpec((BM, D), lambda i: (i, 0)),
                   pl.BlockSpec((8, D), lambda i: (0, 0))],
        out_shape=[jax.ShapeDtypeStruct((N, D), jnp.float32),
                   jax.ShapeDtypeStruct((8, D), jnp.float32)],
        scratch_shapes=[pltpu.VMEM((8, D), jnp.float32)],
    )(alo, ahi, hlo, hhi, dinv16, b2d)


def _norm_body(o_ref, st_ref, g_ref, be_ref, dinv_ref, w_ref,
               hlo_ref, hhi_ref):
    st = st_ref[...]
    m = st[0:1, :] * (1.0 / N)
    var = st[1:2, :] * (1.0 / N) - m * m
    sc = g_ref[...] * lax.rsqrt(var + EPS)
    y = jnp.maximum((o_ref[...] - m) * sc + be_ref[...], 0.0)
    z = y * dinv_ref[...][:, 0:1]
    h = jnp.dot(z, w_ref[...], preferred_element_type=jnp.float32)
    hlo_ref[...] = h[:, :FH]
    hhi_ref[...] = h[:, FH:]


def _norm_mm_call(o, st, g2d, be2d, dinv16, w):
    return pl.pallas_call(
        _norm_body,
        grid=(N // BM,),
        in_specs=[pl.BlockSpec((BM, D), lambda i: (i, 0)),
                  pl.BlockSpec((8, D), lambda i: (0, 0)),
                  pl.BlockSpec((1, D), lambda i: (0, 0)),
                  pl.BlockSpec((1, D), lambda i: (0, 0)),
                  pl.BlockSpec((BM, 16), lambda i: (i, 0)),
                  pl.BlockSpec((D, D), lambda i: (0, 0))],
        out_specs=[pl.BlockSpec((BM, FH), lambda i: (i, 0)),
                   pl.BlockSpec((BM, FH), lambda i: (i, 0))],
        out_shape=[jax.ShapeDtypeStruct((N, FH), jnp.float32),
                   jax.ShapeDtypeStruct((N, FH), jnp.float32)],
    )(o, st, g2d, be2d, dinv16, w)


def _norm_only_body(o_ref, st_ref, g_ref, be_ref, ylo_ref, yhi_ref):
    st = st_ref[...]
    m = st[0:1, :] * (1.0 / N)
    var = st[1:2, :] * (1.0 / N) - m * m
    sc = g_ref[...] * lax.rsqrt(var + EPS)
    y = jnp.maximum((o_ref[...] - m) * sc + be_ref[...], 0.0)
    ylo_ref[...] = y[:, :FH]
    yhi_ref[...] = y[:, FH:]


def _norm_only_call(o, st, g2d, be2d):
    return pl.pallas_call(
        _norm_only_body,
        grid=(N // BM,),
        in_specs=[pl.BlockSpec((BM, D), lambda i: (i, 0)),
                  pl.BlockSpec((8, D), lambda i: (0, 0)),
                  pl.BlockSpec((1, D), lambda i: (0, 0)),
                  pl.BlockSpec((1, D), lambda i: (0, 0))],
        out_specs=[pl.BlockSpec((BM, FH), lambda i: (i, 0)),
                   pl.BlockSpec((BM, FH), lambda i: (i, 0))],
        out_shape=[jax.ShapeDtypeStruct((N, FH), jnp.float32),
                   jax.ShapeDtypeStruct((N, FH), jnp.float32)],
    )(o, st, g2d, be2d)


def _final_body(plo_ref, phi_ref, c_ref, o_ref):
    cnt = jnp.maximum(c_ref[...][:, 0:1], 1.0)
    plo = plo_ref[...]
    phi = phi_ref[...]
    o_ref[:, :FH] = plo / cnt
    o_ref[:, FH:D] = phi / cnt
    o_ref[:, D:D + FH] = plo
    o_ref[:, D + FH:] = phi


def _final_call(plo, phi, cnt):
    return pl.pallas_call(
        _final_body,
        in_specs=[pl.BlockSpec((G, FH), lambda: (0, 0)),
                  pl.BlockSpec((G, FH), lambda: (0, 0)),
                  pl.BlockSpec((G, FH), lambda: (0, 0))],
        out_specs=pl.BlockSpec((G, 2 * D), lambda: (0, 0)),
        out_shape=jax.ShapeDtypeStruct((G, 2 * D), jnp.float32),
    )(plo, phi, cnt)


def kernel(x, edge_index, batch, W1, b1, g1, be1, W2, b2, g2, be2,
           W3, b3, g3, be3):
    src = edge_index[0]
    dst = edge_index[1]
    zeros128 = jnp.zeros((N, FH), jnp.float32)
    zpool = jnp.zeros((G, FH), jnp.float32)
    ones128 = jnp.ones((CH, FH), jnp.float32)
    ones_deg = jnp.ones((C2, FH), jnp.float32)

    dega, degb = _sc_degree_call(dst, zeros128, ones_deg)
    dinv16, hlo, hhi = _t1_call(dega, degb, x, W1)

    for (b, g, be, wn) in ((b1, g1, be1, W2), (b2, g2, be2, W3),
                           (b3, g3, be3, None)):
        alo, ahi = _sc_agg_call(hlo, hhi, src, dst, zeros128)
        o, st = _post_call(alo, ahi, hlo, hhi, dinv16, b.reshape(1, D))
        if wn is not None:
            hlo, hhi = _norm_mm_call(o, st, g.reshape(1, D),
                                     be.reshape(1, D), dinv16, wn)
        else:
            ylo, yhi = _norm_only_call(o, st, g.reshape(1, D),
                                       be.reshape(1, D))

    plo, phi, cnt = _sc_pool_call(ylo, yhi, batch, zpool, ones128)
    return _final_call(plo, phi, cnt)
